# Initial kernel scaffold; baseline (speedup 1.0000x reference)
import jax
import jax.numpy as jnp
from jax.experimental import pallas as pl


def kernel(x, edge_index, edge_weight, edge_mask, Ws0, bs0, Wn0, bn0, Ws1, bs1, Wn1, bn1, Ws2, bs2, Wn2, bn2, Ws3, bs3, Wn3, bn3, lin1_W, lin1_b, lin2_W, lin2_b):
    raise NotImplementedError("write your pallas kernel here")


# speedup vs baseline: 10.8620x; 10.8620x over previous
import functools

import jax
import jax.numpy as jnp
from jax import lax
from jax.experimental import pallas as pl
from jax.experimental.pallas import tpu as pltpu
from jax.experimental.pallas import tpu_sc as plsc

N = 10000
E = 320000
DIN = 128
DH = 32
NQ = 2048
NPAIR = NQ // 2

NC = 2
NS = 16
NW = NC * NS
CH = 128
ROWS_PER_TEC = 80
EPAD = NW * ROWS_PER_TEC * CH
NROWS = EPAD // CH
NPAD = 10112
RPT = NPAD // NS

_f32 = jnp.float32


def _sc_layer_body(with_deg, hn_hbm, src_hbm, dst_hbm, ew_hbm, *refs):
    if with_deg:
        agg_out, deg_out, src_v, dst_v, ew_v, rows_v, ones_v, zvec_v, agg_sh, deg_sh, sem = refs
    else:
        agg_out, src_v, dst_v, ew_v, rows_v, ones_v, zvec_v, agg_sh, sem = refs

    c = lax.axis_index("c")
    s = lax.axis_index("s")
    wid = s * NC + c

    zero16 = jnp.zeros((16,), _f32)

    def _zero_rows(i, _):
        rows_v[i, 0:16] = zero16
        rows_v[i, 16:32] = zero16
        return 0

    lax.fori_loop(0, CH, _zero_rows, 0)

    def _fill_ones(i, _):
        ones_v[pl.ds(i * 16, 16)] = jnp.ones((16,), _f32)
        return 0

    lax.fori_loop(0, CH // 16, _fill_ones, 0)

    if with_deg:
        def _zero_zvec(i, _):
            zvec_v[pl.ds(i * 16, 16)] = zero16
            return 0

        lax.fori_loop(0, 640 // 16, _zero_zvec, 0)

    base_n = s * RPT
    for k in range(4):
        pltpu.sync_copy(rows_v, agg_sh.at[pl.ds(base_n + k * CH, CH)])
    pltpu.sync_copy(rows_v.at[pl.ds(0, RPT - 4 * CH)],
                    agg_sh.at[pl.ds(base_n + 4 * CH, RPT - 4 * CH)])
    if with_deg:
        pltpu.sync_copy(zvec_v.at[pl.ds(0, RPT)], deg_sh.at[pl.ds(base_n, RPT)])

    plsc.subcore_barrier()

    row0 = wid * ROWS_PER_TEC
    pltpu.sync_copy(src_hbm.at[pl.ds(row0, ROWS_PER_TEC)], src_v)
    pltpu.sync_copy(dst_hbm.at[pl.ds(row0, ROWS_PER_TEC)], dst_v)
    pltpu.sync_copy(ew_hbm.at[pl.ds(row0, ROWS_PER_TEC)], ew_v)

    def _edge_window(j, _):
        idx_row = src_v.at[j]
        pltpu.async_copy(hn_hbm.at[idx_row], rows_v, sem).wait()

        dn = lax.GatherDimensionNumbers(offset_dims=(), collapsed_slice_dims=(0,),
                                        start_index_map=(0,))

        def _scale_group(g, _):
            ewv = ew_v[j, pl.ds(g * 16, 16)]
            for e in range(16):
                ewb = lax.gather(ewv, jnp.full((16, 1), e, jnp.int32), dn, (1,),
                                 mode=lax.GatherScatterMode.PROMISE_IN_BOUNDS)
                r = g * 16 + e
                rows_v[r, 0:16] = rows_v[r, 0:16] * ewb
                rows_v[r, 16:32] = rows_v[r, 16:32] * ewb
            return 0

        lax.fori_loop(0, CH // 16, _scale_group, 0)

        dst_row = dst_v.at[j]
        pltpu.sync_copy(rows_v, agg_sh.at[dst_row], add=True)
        if with_deg:
            pltpu.sync_copy(ones_v, deg_sh.at[dst_row], add=True)
        return 0

    lax.fori_loop(0, ROWS_PER_TEC, _edge_window, 0)

    plsc.subcore_barrier()

    pltpu.sync_copy(agg_sh.at[pl.ds(base_n, RPT)], agg_out.at[c, pl.ds(base_n, RPT)])
    if with_deg:
        pltpu.sync_copy(deg_sh.at[pl.ds(base_n, RPT)],
                        deg_out.at[pl.ds(c * NPAD + base_n, RPT)])


def _make_sc_layer(with_deg, interpret=False):
    mesh = plsc.VectorSubcoreMesh(core_axis_name="c", subcore_axis_name="s",
                                  num_cores=NC, num_subcores=NS)
    out_type = [jax.ShapeDtypeStruct((NC, NPAD, DH), _f32)]
    if with_deg:
        out_type.append(jax.ShapeDtypeStruct((NC * NPAD,), _f32))
    scratch = [
        pltpu.VMEM((ROWS_PER_TEC, CH), jnp.int32),
        pltpu.VMEM((ROWS_PER_TEC, CH), jnp.int32),
        pltpu.VMEM((ROWS_PER_TEC, CH), _f32),
        pltpu.VMEM((CH, DH), _f32),
        pltpu.VMEM((CH,), _f32),
        pltpu.VMEM((640,), _f32),
        pltpu.VMEM_SHARED((NPAD, DH), _f32),
    ]
    if with_deg:
        scratch.append(pltpu.VMEM_SHARED((NPAD,), _f32))
    scratch.append(pltpu.SemaphoreType.DMA)
    return pl.kernel(functools.partial(_sc_layer_body, with_deg),
                     out_type=tuple(out_type), mesh=mesh,
                     scratch_types=tuple(scratch),
                     compiler_params=pltpu.CompilerParams(use_tc_tiling_on_sc=False),
                     interpret=interpret)


_BR = 2000


def _dotT(a, w):
    return lax.dot_general(a, w, (((1,), (1,)), ((), ())),
                           preferred_element_type=_f32)


def _tc_pre_body(x_ref, wn_ref, ws_ref, b_ref, hn_ref, hs_ref):
    xb = x_ref[...]
    hn_ref[...] = _dotT(xb, wn_ref[...])
    hs_ref[...] = _dotT(xb, ws_ref[...]) + b_ref[...]


def _tc_pre(x, wn, ws, b, interpret=False):
    grid = (N // _BR,)
    return pl.pallas_call(
        _tc_pre_body,
        grid=grid,
        in_specs=[
            pl.BlockSpec((_BR, DIN), lambda i: (i, 0)),
            pl.BlockSpec((DH, DIN), lambda i: (0, 0)),
            pl.BlockSpec((DH, DIN), lambda i: (0, 0)),
            pl.BlockSpec((1, DH), lambda i: (0, 0)),
        ],
        out_specs=[
            pl.BlockSpec((_BR, DH), lambda i: (i, 0)),
            pl.BlockSpec((_BR, DH), lambda i: (i, 0)),
        ],
        out_shape=[
            jax.ShapeDtypeStruct((N, DH), _f32),
            jax.ShapeDtypeStruct((N, DH), _f32),
        ],
        interpret=interpret,
    )(x, wn, ws, b)


def _tc_mid_body(hs_ref, a0_ref, a1_ref, d0_ref, d1_ref, wn_ref, ws_ref, b_ref,
                 h_ref, hn_ref, hs_out_ref):
    deg = jnp.maximum(d0_ref[...] + d1_ref[...], 1.0)
    h = jnp.tanh(hs_ref[...] + (a0_ref[...] + a1_ref[...]) / deg)
    h_ref[...] = h
    hn_ref[...] = _dotT(h, wn_ref[...])
    hs_out_ref[...] = _dotT(h, ws_ref[...]) + b_ref[...]


def _tc_mid(hs, a0, a1, d0, d1, wn, ws, b, interpret=False):
    grid = (N // _BR,)
    return pl.pallas_call(
        _tc_mid_body,
        grid=grid,
        in_specs=[
            pl.BlockSpec((_BR, DH), lambda i: (i, 0)),
            pl.BlockSpec((_BR, DH), lambda i: (i, 0)),
            pl.BlockSpec((_BR, DH), lambda i: (i, 0)),
            pl.BlockSpec((_BR, 1), lambda i: (i, 0)),
            pl.BlockSpec((_BR, 1), lambda i: (i, 0)),
            pl.BlockSpec((DH, DH), lambda i: (0, 0)),
            pl.BlockSpec((DH, DH), lambda i: (0, 0)),
            pl.BlockSpec((1, DH), lambda i: (0, 0)),
        ],
        out_specs=[
            pl.BlockSpec((_BR, DH), lambda i: (i, 0)),
            pl.BlockSpec((_BR, DH), lambda i: (i, 0)),
            pl.BlockSpec((_BR, DH), lambda i: (i, 0)),
        ],
        out_shape=[
            jax.ShapeDtypeStruct((N, DH), _f32),
            jax.ShapeDtypeStruct((N, DH), _f32),
            jax.ShapeDtypeStruct((N, DH), _f32),
        ],
        interpret=interpret,
    )(hs, a0, a1, d0, d1, wn, ws, b)


def _tc_head_body(hs3e_ref, hs3o_ref, a0e_ref, a0o_ref, a1e_ref, a1o_ref,
                  d0e_ref, d0o_ref, d1e_ref, d1o_ref,
                  h1e_ref, h2e_ref, h3e_ref, h1o_ref, h2o_ref, h3o_ref,
                  qe_ref, qo_ref, wa_ref, wb_ref, b1_ref, w2_ref, b2_ref,
                  out_ref):
    dege = jnp.maximum(d0e_ref[...] + d1e_ref[...], 1.0)
    dego = jnp.maximum(d0o_ref[...] + d1o_ref[...], 1.0)
    h4e = jnp.tanh(hs3e_ref[...] + (a0e_ref[...] + a1e_ref[...]) / dege)
    h4o = jnp.tanh(hs3o_ref[...] + (a0o_ref[...] + a1o_ref[...]) / dego)
    qe = qe_ref[...] == 1.0
    qo = qo_ref[...] == 1.0
    he = [h1e_ref[...], h2e_ref[...], h3e_ref[...], h4e]
    ho = [h1o_ref[...], h2o_ref[...], h3o_ref[...], h4o]
    y = b1_ref[...]
    for i in range(4):
        me = jnp.where(qe, he[i], 0.0)
        mo = jnp.where(qo, ho[i], 0.0)
        y = y + _dotT(me, wa_ref[i])
        y = y + _dotT(mo, wb_ref[i])
    y = jnp.maximum(y, 0.0)
    z = jnp.sum(y * w2_ref[...], axis=1, keepdims=True) + b2_ref[0, 0]
    out_ref[...] = jax.nn.sigmoid(z)


def _tc_head(args, interpret=False):
    full32 = pl.BlockSpec((NPAIR, DH), lambda: (0, 0))
    full1 = pl.BlockSpec((NPAIR, 1), lambda: (0, 0))
    return pl.pallas_call(
        _tc_head_body,
        in_specs=[full32, full32, full32, full32, full32, full32,
                  full1, full1, full1, full1,
                  full32, full32, full32, full32, full32, full32,
                  full1, full1,
                  pl.BlockSpec((4, DIN, DH), lambda: (0, 0, 0)),
                  pl.BlockSpec((4, DIN, DH), lambda: (0, 0, 0)),
                  pl.BlockSpec((1, DIN), lambda: (0, 0)),
                  pl.BlockSpec((1, DIN), lambda: (0, 0)),
                  pl.BlockSpec(memory_space=pltpu.SMEM)],
        out_specs=pl.BlockSpec((NPAIR, 1), lambda: (0, 0)),
        out_shape=jax.ShapeDtypeStruct((NPAIR, 1), _f32),
        interpret=interpret,
    )(*args)


def kernel(x, edge_index, edge_weight, edge_mask,
           Ws0, bs0, Wn0, bn0, Ws1, bs1, Wn1, bn1,
           Ws2, bs2, Wn2, bn2, Ws3, bs3, Wn3, bn3,
           lin1_W, lin1_b, lin2_W, lin2_b):
    src = edge_index[0]
    dst = edge_index[1]
    ew = edge_weight * edge_mask

    npad_e = EPAD - E
    pad_i = jnp.arange(npad_e, dtype=jnp.int32)
    src_p = jnp.concatenate([src, pad_i % 16]).reshape(NROWS, CH)
    dst_p = jnp.concatenate([dst, N + pad_i % (NPAD - N)]).reshape(NROWS, CH)
    ew_p = jnp.concatenate([ew, jnp.zeros((npad_e,), _f32)]).reshape(NROWS, CH)

    sc_deg = _make_sc_layer(True)
    sc_nodeg = _make_sc_layer(False)

    params = [(Ws0, bs0, Wn0, bn0), (Ws1, bs1, Wn1, bn1),
              (Ws2, bs2, Wn2, bn2), (Ws3, bs3, Wn3, bn3)]

    hn, hs = _tc_pre(x, Wn0, Ws0, (bs0 + bn0).reshape(1, DH))

    states = []
    d0 = d1 = None
    for l in range(4):
        if l == 0:
            agg, deg = sc_deg(hn, src_p, dst_p, ew_p)
            d0 = deg[:N].reshape(N, 1)
            d1 = deg[NPAD:NPAD + N].reshape(N, 1)
        else:
            (agg,) = sc_nodeg(hn, src_p, dst_p, ew_p)
        a0 = agg[0, :N]
        a1 = agg[1, :N]
        if l < 3:
            Ws_n, bs_n, Wn_n, bn_n = params[l + 1]
            h, hn, hs = _tc_mid(hs, a0, a1, d0, d1, Wn_n, Ws_n,
                                (bs_n + bn_n).reshape(1, DH))
            states.append(h)
        else:
            hs3e, hs3o = hs[:NQ:2], hs[1:NQ:2]
            a0e, a0o = a0[:NQ:2], a0[1:NQ:2]
            a1e, a1o = a1[:NQ:2], a1[1:NQ:2]
            d0e, d0o = d0[:NQ:2], d0[1:NQ:2]
            d1e, d1o = d1[:NQ:2], d1[1:NQ:2]
            h_e = [st[:NQ:2] for st in states]
            h_o = [st[1:NQ:2] for st in states]
            qcol = x[:NQ, 0:1]
            qe, qo = qcol[::2], qcol[1::2]
            wa = jnp.stack([lin1_W[:, 32 * i:32 * i + 32] for i in range(4)])
            wb = jnp.stack([lin1_W[:, 128 + 32 * i:128 + 32 * i + 32]
                            for i in range(4)])
            out = _tc_head((hs3e, hs3o, a0e, a0o, a1e, a1o,
                            d0e, d0o, d1e, d1o,
                            h_e[0], h_e[1], h_e[2], h_o[0], h_o[1], h_o[2],
                            qe, qo, wa, wb,
                            lin1_b.reshape(1, DIN), lin2_W,
                            lin2_b.reshape(1, 1)))
    return out


# speedup vs baseline: 17.4416x; 1.6057x over previous
import functools

import jax
import jax.numpy as jnp
from jax import lax
from jax.experimental import pallas as pl
from jax.experimental.pallas import tpu as pltpu
from jax.experimental.pallas import tpu_sc as plsc

N = 10000
E = 320000
DIN = 128
DH = 32
NQ = 2048
NPAIR = NQ // 2

NC = 2
NS = 16
NW = NC * NS
CH = 128
ROWS_PER_TEC = 80
EPAD = NW * ROWS_PER_TEC * CH
NROWS = EPAD // CH
NPAD = 10112
RPT = NPAD // NS

_f32 = jnp.float32


NBUF = 4


def _sc_layer_body(with_deg, hn_hbm, src_hbm, dst_hbm, ew_hbm, *refs):
    if with_deg:
        (agg_out, deg_out, src_v, dst_v, ew_v, g0, g1, g2, g3, s0, s1, s2, s3,
         ones_v, zvec_v, agg_sh, deg_sh,
         gm0, gm1, gm2, gm3, sm0, sm1, sm2, sm3, sem_d) = refs
    else:
        (agg_out, src_v, dst_v, ew_v, g0, g1, g2, g3, s0, s1, s2, s3,
         ones_v, zvec_v, agg_sh,
         gm0, gm1, gm2, gm3, sm0, sm1, sm2, sm3) = refs
    gbuf = [g0, g1, g2, g3]
    sbuf = [s0, s1, s2, s3]
    gsem = [gm0, gm1, gm2, gm3]
    ssem = [sm0, sm1, sm2, sm3]

    c = lax.axis_index("c")
    s = lax.axis_index("s")
    wid = s * NC + c

    zero16 = jnp.zeros((16,), _f32)

    def _zero_rows(i, _):
        s0[i, 0:16] = zero16
        s0[i, 16:32] = zero16
        return 0

    lax.fori_loop(0, CH, _zero_rows, 0)

    def _fill_ones(i, _):
        ones_v[pl.ds(i * 16, 16)] = jnp.ones((16,), _f32)
        return 0

    lax.fori_loop(0, CH // 16, _fill_ones, 0)

    if with_deg:
        def _zero_zvec(i, _):
            zvec_v[pl.ds(i * 16, 16)] = zero16
            return 0

        lax.fori_loop(0, 640 // 16, _zero_zvec, 0)

    base_n = s * RPT
    for k in range(4):
        pltpu.sync_copy(s0, agg_sh.at[pl.ds(base_n + k * CH, CH)])
    pltpu.sync_copy(s0.at[pl.ds(0, RPT - 4 * CH)],
                    agg_sh.at[pl.ds(base_n + 4 * CH, RPT - 4 * CH)])
    if with_deg:
        pltpu.sync_copy(zvec_v.at[pl.ds(0, RPT)], deg_sh.at[pl.ds(base_n, RPT)])

    plsc.subcore_barrier()

    row0 = wid * ROWS_PER_TEC
    pltpu.sync_copy(src_hbm.at[pl.ds(row0, ROWS_PER_TEC)], src_v)
    pltpu.sync_copy(dst_hbm.at[pl.ds(row0, ROWS_PER_TEC)], dst_v)
    pltpu.sync_copy(ew_hbm.at[pl.ds(row0, ROWS_PER_TEC)], ew_v)

    dn = lax.GatherDimensionNumbers(offset_dims=(), collapsed_slice_dims=(0,),
                                    start_index_map=(0,))

    def _gather(win, b):
        return pltpu.make_async_copy(hn_hbm.at[src_v.at[win]], gbuf[b], gsem[b])

    def _scatter_start(b, win):
        pltpu.async_copy(sbuf[b], agg_sh.at[dst_v.at[win]], ssem[b], add=True)

    def _scatter_wait(b):
        pltpu.make_async_copy(sbuf[b], agg_sh.at[dst_v.at[0]], ssem[b]).wait()

    for b in range(NBUF):
        _gather(b, b).start()

    def _outer(i, _):
        j0 = i * NBUF
        for b in range(NBUF):
            j = j0 + b
            _gather(j, b).wait()

            @pl.when(j >= NBUF)
            def _():
                _scatter_wait(b)

            def _scale_group(g_, _c):
                ewv = ew_v[j, pl.ds(g_ * 16, 16)]
                for e in range(16):
                    ewb = lax.gather(ewv, jnp.full((16, 1), e, jnp.int32), dn,
                                     (1,),
                                     mode=lax.GatherScatterMode.PROMISE_IN_BOUNDS)
                    r = g_ * 16 + e
                    sbuf[b][r, 0:16] = gbuf[b][r, 0:16] * ewb
                    sbuf[b][r, 16:32] = gbuf[b][r, 16:32] * ewb
                return 0

            lax.fori_loop(0, CH // 16, _scale_group, 0)

            _scatter_start(b, j)
            if with_deg:
                pltpu.async_copy(ones_v, deg_sh.at[dst_v.at[j]], sem_d, add=True)

                @pl.when(j >= NBUF)
                def _():
                    pltpu.make_async_copy(ones_v, deg_sh.at[dst_v.at[0]],
                                          sem_d).wait()

            @pl.when(j + NBUF < ROWS_PER_TEC)
            def _():
                _gather(j + NBUF, b).start()
        return 0

    lax.fori_loop(0, ROWS_PER_TEC // NBUF, _outer, 0)

    for b in range(NBUF):
        _scatter_wait(b)

    if with_deg:
        for _b in range(NBUF):
            pltpu.make_async_copy(ones_v, deg_sh.at[dst_v.at[0]], sem_d).wait()

    plsc.subcore_barrier()

    pltpu.sync_copy(agg_sh.at[pl.ds(base_n, RPT)], agg_out.at[c, pl.ds(base_n, RPT)])
    if with_deg:
        pltpu.sync_copy(deg_sh.at[pl.ds(base_n, RPT)],
                        deg_out.at[pl.ds(c * NPAD + base_n, RPT)])


def _make_sc_layer(with_deg, interpret=False):
    mesh = plsc.VectorSubcoreMesh(core_axis_name="c", subcore_axis_name="s",
                                  num_cores=NC, num_subcores=NS)
    out_type = [jax.ShapeDtypeStruct((NC, NPAD, DH), _f32)]
    if with_deg:
        out_type.append(jax.ShapeDtypeStruct((NC * NPAD,), _f32))
    scratch = [
        pltpu.VMEM((ROWS_PER_TEC, CH), jnp.int32),
        pltpu.VMEM((ROWS_PER_TEC, CH), jnp.int32),
        pltpu.VMEM((ROWS_PER_TEC, CH), _f32),
    ]
    scratch += [pltpu.VMEM((CH, DH), _f32) for _ in range(2 * NBUF)]
    scratch += [
        pltpu.VMEM((CH,), _f32),
        pltpu.VMEM((640,), _f32),
        pltpu.VMEM_SHARED((NPAD, DH), _f32),
    ]
    if with_deg:
        scratch.append(pltpu.VMEM_SHARED((NPAD,), _f32))
    scratch += [pltpu.SemaphoreType.DMA for _ in range(2 * NBUF)]
    if with_deg:
        scratch.append(pltpu.SemaphoreType.DMA)
    return pl.kernel(functools.partial(_sc_layer_body, with_deg),
                     out_type=tuple(out_type), mesh=mesh,
                     scratch_types=tuple(scratch),
                     compiler_params=pltpu.CompilerParams(use_tc_tiling_on_sc=False),
                     interpret=interpret)


_BR = 2000


def _dotT(a, w):
    return lax.dot_general(a, w, (((1,), (1,)), ((), ())),
                           preferred_element_type=_f32)


def _tc_pre_body(x_ref, wn_ref, ws_ref, b_ref, hn_ref, hs_ref):
    xb = x_ref[...]
    hn_ref[...] = _dotT(xb, wn_ref[...])
    hs_ref[...] = _dotT(xb, ws_ref[...]) + b_ref[...]


def _tc_pre(x, wn, ws, b, interpret=False):
    grid = (N // _BR,)
    return pl.pallas_call(
        _tc_pre_body,
        grid=grid,
        in_specs=[
            pl.BlockSpec((_BR, DIN), lambda i: (i, 0)),
            pl.BlockSpec((DH, DIN), lambda i: (0, 0)),
            pl.BlockSpec((DH, DIN), lambda i: (0, 0)),
            pl.BlockSpec((1, DH), lambda i: (0, 0)),
        ],
        out_specs=[
            pl.BlockSpec((_BR, DH), lambda i: (i, 0)),
            pl.BlockSpec((_BR, DH), lambda i: (i, 0)),
        ],
        out_shape=[
            jax.ShapeDtypeStruct((N, DH), _f32),
            jax.ShapeDtypeStruct((N, DH), _f32),
        ],
        interpret=interpret,
    )(x, wn, ws, b)


def _tc_mid_body(hs_ref, a0_ref, a1_ref, d0_ref, d1_ref, wn_ref, ws_ref, b_ref,
                 h_ref, hn_ref, hs_out_ref):
    deg = jnp.maximum(d0_ref[...] + d1_ref[...], 1.0)
    h = jnp.tanh(hs_ref[...] + (a0_ref[...] + a1_ref[...]) / deg)
    h_ref[...] = h
    hn_ref[...] = _dotT(h, wn_ref[...])
    hs_out_ref[...] = _dotT(h, ws_ref[...]) + b_ref[...]


def _tc_mid(hs, a0, a1, d0, d1, wn, ws, b, interpret=False):
    grid = (N // _BR,)
    return pl.pallas_call(
        _tc_mid_body,
        grid=grid,
        in_specs=[
            pl.BlockSpec((_BR, DH), lambda i: (i, 0)),
            pl.BlockSpec((_BR, DH), lambda i: (i, 0)),
            pl.BlockSpec((_BR, DH), lambda i: (i, 0)),
            pl.BlockSpec((_BR, 1), lambda i: (i, 0)),
            pl.BlockSpec((_BR, 1), lambda i: (i, 0)),
            pl.BlockSpec((DH, DH), lambda i: (0, 0)),
            pl.BlockSpec((DH, DH), lambda i: (0, 0)),
            pl.BlockSpec((1, DH), lambda i: (0, 0)),
        ],
        out_specs=[
            pl.BlockSpec((_BR, DH), lambda i: (i, 0)),
            pl.BlockSpec((_BR, DH), lambda i: (i, 0)),
            pl.BlockSpec((_BR, DH), lambda i: (i, 0)),
        ],
        out_shape=[
            jax.ShapeDtypeStruct((N, DH), _f32),
            jax.ShapeDtypeStruct((N, DH), _f32),
            jax.ShapeDtypeStruct((N, DH), _f32),
        ],
        interpret=interpret,
    )(hs, a0, a1, d0, d1, wn, ws, b)


def _tc_head_body(hs3e_ref, hs3o_ref, a0e_ref, a0o_ref, a1e_ref, a1o_ref,
                  d0e_ref, d0o_ref, d1e_ref, d1o_ref,
                  h1e_ref, h2e_ref, h3e_ref, h1o_ref, h2o_ref, h3o_ref,
                  qe_ref, qo_ref, wa_ref, wb_ref, b1_ref, w2_ref, b2_ref,
                  out_ref):
    dege = jnp.maximum(d0e_ref[...] + d1e_ref[...], 1.0)
    dego = jnp.maximum(d0o_ref[...] + d1o_ref[...], 1.0)
    h4e = jnp.tanh(hs3e_ref[...] + (a0e_ref[...] + a1e_ref[...]) / dege)
    h4o = jnp.tanh(hs3o_ref[...] + (a0o_ref[...] + a1o_ref[...]) / dego)
    qe = qe_ref[...] == 1.0
    qo = qo_ref[...] == 1.0
    he = [h1e_ref[...], h2e_ref[...], h3e_ref[...], h4e]
    ho = [h1o_ref[...], h2o_ref[...], h3o_ref[...], h4o]
    y = b1_ref[...]
    for i in range(4):
        me = jnp.where(qe, he[i], 0.0)
        mo = jnp.where(qo, ho[i], 0.0)
        y = y + _dotT(me, wa_ref[i])
        y = y + _dotT(mo, wb_ref[i])
    y = jnp.maximum(y, 0.0)
    z = jnp.sum(y * w2_ref[...], axis=1, keepdims=True) + b2_ref[0, 0]
    out_ref[...] = jax.nn.sigmoid(z)


def _tc_head(args, interpret=False):
    full32 = pl.BlockSpec((NPAIR, DH), lambda: (0, 0))
    full1 = pl.BlockSpec((NPAIR, 1), lambda: (0, 0))
    return pl.pallas_call(
        _tc_head_body,
        in_specs=[full32, full32, full32, full32, full32, full32,
                  full1, full1, full1, full1,
                  full32, full32, full32, full32, full32, full32,
                  full1, full1,
                  pl.BlockSpec((4, DIN, DH), lambda: (0, 0, 0)),
                  pl.BlockSpec((4, DIN, DH), lambda: (0, 0, 0)),
                  pl.BlockSpec((1, DIN), lambda: (0, 0)),
                  pl.BlockSpec((1, DIN), lambda: (0, 0)),
                  pl.BlockSpec(memory_space=pltpu.SMEM)],
        out_specs=pl.BlockSpec((NPAIR, 1), lambda: (0, 0)),
        out_shape=jax.ShapeDtypeStruct((NPAIR, 1), _f32),
        interpret=interpret,
    )(*args)


def kernel(x, edge_index, edge_weight, edge_mask,
           Ws0, bs0, Wn0, bn0, Ws1, bs1, Wn1, bn1,
           Ws2, bs2, Wn2, bn2, Ws3, bs3, Wn3, bn3,
           lin1_W, lin1_b, lin2_W, lin2_b):
    src = edge_index[0]
    dst = edge_index[1]
    ew = edge_weight * edge_mask

    npad_e = EPAD - E
    pad_i = jnp.arange(npad_e, dtype=jnp.int32)
    src_p = jnp.concatenate([src, pad_i % 16]).reshape(NROWS, CH)
    dst_p = jnp.concatenate([dst, N + pad_i % (NPAD - N)]).reshape(NROWS, CH)
    ew_p = jnp.concatenate([ew, jnp.zeros((npad_e,), _f32)]).reshape(NROWS, CH)

    sc_deg = _make_sc_layer(True)
    sc_nodeg = _make_sc_layer(False)

    params = [(Ws0, bs0, Wn0, bn0), (Ws1, bs1, Wn1, bn1),
              (Ws2, bs2, Wn2, bn2), (Ws3, bs3, Wn3, bn3)]

    hn, hs = _tc_pre(x, Wn0, Ws0, (bs0 + bn0).reshape(1, DH))

    states = []
    d0 = d1 = None
    for l in range(4):
        if l == 0:
            agg, deg = sc_deg(hn, src_p, dst_p, ew_p)
            d0 = deg[:N].reshape(N, 1)
            d1 = deg[NPAD:NPAD + N].reshape(N, 1)
        else:
            (agg,) = sc_nodeg(hn, src_p, dst_p, ew_p)
        a0 = agg[0, :N]
        a1 = agg[1, :N]
        if l < 3:
            Ws_n, bs_n, Wn_n, bn_n = params[l + 1]
            h, hn, hs = _tc_mid(hs, a0, a1, d0, d1, Wn_n, Ws_n,
                                (bs_n + bn_n).reshape(1, DH))
            states.append(h)
        else:
            hs3e, hs3o = hs[:NQ:2], hs[1:NQ:2]
            a0e, a0o = a0[:NQ:2], a0[1:NQ:2]
            a1e, a1o = a1[:NQ:2], a1[1:NQ:2]
            d0e, d0o = d0[:NQ:2], d0[1:NQ:2]
            d1e, d1o = d1[:NQ:2], d1[1:NQ:2]
            h_e = [st[:NQ:2] for st in states]
            h_o = [st[1:NQ:2] for st in states]
            qcol = x[:NQ, 0:1]
            qe, qo = qcol[::2], qcol[1::2]
            wa = jnp.stack([lin1_W[:, 32 * i:32 * i + 32] for i in range(4)])
            wb = jnp.stack([lin1_W[:, 128 + 32 * i:128 + 32 * i + 32]
                            for i in range(4)])
            out = _tc_head((hs3e, hs3o, a0e, a0o, a1e, a1o,
                            d0e, d0o, d1e, d1o,
                            h_e[0], h_e[1], h_e[2], h_o[0], h_o[1], h_o[2],
                            qe, qo, wa, wb,
                            lin1_b.reshape(1, DIN), lin2_W,
                            lin2_b.reshape(1, 1)))
    return out


# speedup vs baseline: 18.5417x; 1.0631x over previous
import functools

import jax
import jax.numpy as jnp
from jax import lax
from jax.experimental import pallas as pl
from jax.experimental.pallas import tpu as pltpu
from jax.experimental.pallas import tpu_sc as plsc

N = 10000
E = 320000
DIN = 128
DH = 32
NQ = 2048
NPAIR = NQ // 2

NC = 2
NS = 16
NW = NC * NS
CH = 128
ROWS_PER_TEC = 80
EPAD = NW * ROWS_PER_TEC * CH
NROWS = EPAD // CH
NPAD = 10112
RPT = NPAD // NS

_f32 = jnp.float32


NBUF = 8


def _sc_layer_body(with_deg, hn_hbm, src_hbm, dst_hbm, ew_hbm, *refs):
    if with_deg:
        agg_out, deg_out = refs[0], refs[1]
        rest = refs[2:]
    else:
        agg_out = refs[0]
        rest = refs[1:]
    src_v, dst_v, ew_v = rest[0], rest[1], rest[2]
    gbuf = list(rest[3:3 + NBUF])
    sbuf = list(rest[3 + NBUF:3 + 2 * NBUF])
    k = 3 + 2 * NBUF
    ones_v, zvec_v, agg_sh = rest[k], rest[k + 1], rest[k + 2]
    k += 3
    if with_deg:
        deg_sh = rest[k]
        k += 1
    gsem = list(rest[k:k + NBUF])
    ssem = list(rest[k + NBUF:k + 2 * NBUF])
    if with_deg:
        sem_d = rest[k + 2 * NBUF]
    s0 = sbuf[0]

    c = lax.axis_index("c")
    s = lax.axis_index("s")
    wid = s * NC + c

    zero16 = jnp.zeros((16,), _f32)

    def _zero_rows(i, _):
        s0[i, 0:16] = zero16
        s0[i, 16:32] = zero16
        return 0

    lax.fori_loop(0, CH, _zero_rows, 0)

    def _fill_ones(i, _):
        ones_v[pl.ds(i * 16, 16)] = jnp.ones((16,), _f32)
        return 0

    lax.fori_loop(0, CH // 16, _fill_ones, 0)

    if with_deg:
        def _zero_zvec(i, _):
            zvec_v[pl.ds(i * 16, 16)] = zero16
            return 0

        lax.fori_loop(0, 640 // 16, _zero_zvec, 0)

    base_n = s * RPT
    for k in range(4):
        pltpu.sync_copy(s0, agg_sh.at[pl.ds(base_n + k * CH, CH)])
    pltpu.sync_copy(s0.at[pl.ds(0, RPT - 4 * CH)],
                    agg_sh.at[pl.ds(base_n + 4 * CH, RPT - 4 * CH)])
    if with_deg:
        pltpu.sync_copy(zvec_v.at[pl.ds(0, RPT)], deg_sh.at[pl.ds(base_n, RPT)])

    plsc.subcore_barrier()

    row0 = wid * ROWS_PER_TEC
    pltpu.sync_copy(src_hbm.at[pl.ds(row0, ROWS_PER_TEC)], src_v)
    pltpu.sync_copy(dst_hbm.at[pl.ds(row0, ROWS_PER_TEC)], dst_v)
    pltpu.sync_copy(ew_hbm.at[pl.ds(row0, ROWS_PER_TEC)], ew_v)

    dn = lax.GatherDimensionNumbers(offset_dims=(), collapsed_slice_dims=(0,),
                                    start_index_map=(0,))

    def _gather(win, b):
        return pltpu.make_async_copy(hn_hbm.at[src_v.at[win]], gbuf[b], gsem[b])

    def _scatter_start(b, win):
        pltpu.async_copy(sbuf[b], agg_sh.at[dst_v.at[win]], ssem[b], add=True)

    def _scatter_wait(b):
        pltpu.make_async_copy(sbuf[b], agg_sh.at[dst_v.at[0]], ssem[b]).wait()

    for b in range(NBUF):
        _gather(b, b).start()

    def _outer(i, _):
        j0 = i * NBUF
        for b in range(NBUF):
            j = j0 + b
            _gather(j, b).wait()

            @pl.when(j >= NBUF)
            def _():
                _scatter_wait(b)

            def _scale_group(g_, _c):
                ewv = ew_v[j, pl.ds(g_ * 16, 16)]
                for e in range(16):
                    ewb = lax.gather(ewv, jnp.full((16, 1), e, jnp.int32), dn,
                                     (1,),
                                     mode=lax.GatherScatterMode.PROMISE_IN_BOUNDS)
                    r = g_ * 16 + e
                    sbuf[b][r, 0:16] = gbuf[b][r, 0:16] * ewb
                    sbuf[b][r, 16:32] = gbuf[b][r, 16:32] * ewb
                return 0

            lax.fori_loop(0, CH // 16, _scale_group, 0)

            _scatter_start(b, j)
            if with_deg:
                pltpu.async_copy(ones_v, deg_sh.at[dst_v.at[j]], sem_d, add=True)

                @pl.when(j >= NBUF)
                def _():
                    pltpu.make_async_copy(ones_v, deg_sh.at[dst_v.at[0]],
                                          sem_d).wait()

            @pl.when(j + NBUF < ROWS_PER_TEC)
            def _():
                _gather(j + NBUF, b).start()
        return 0

    lax.fori_loop(0, ROWS_PER_TEC // NBUF, _outer, 0)

    for b in range(NBUF):
        _scatter_wait(b)

    if with_deg:
        for _b in range(NBUF):
            pltpu.make_async_copy(ones_v, deg_sh.at[dst_v.at[0]], sem_d).wait()

    plsc.subcore_barrier()

    pltpu.sync_copy(agg_sh.at[pl.ds(base_n, RPT)], agg_out.at[c, pl.ds(base_n, RPT)])
    if with_deg:
        pltpu.sync_copy(deg_sh.at[pl.ds(base_n, RPT)],
                        deg_out.at[pl.ds(c * NPAD + base_n, RPT)])


def _make_sc_layer(with_deg, interpret=False):
    mesh = plsc.VectorSubcoreMesh(core_axis_name="c", subcore_axis_name="s",
                                  num_cores=NC, num_subcores=NS)
    out_type = [jax.ShapeDtypeStruct((NC, NPAD, DH), _f32)]
    if with_deg:
        out_type.append(jax.ShapeDtypeStruct((NC * NPAD,), _f32))
    scratch = [
        pltpu.VMEM((ROWS_PER_TEC, CH), jnp.int32),
        pltpu.VMEM((ROWS_PER_TEC, CH), jnp.int32),
        pltpu.VMEM((ROWS_PER_TEC, CH), _f32),
    ]
    assert ROWS_PER_TEC % NBUF == 0
    scratch += [pltpu.VMEM((CH, DH), _f32) for _ in range(2 * NBUF)]
    scratch += [
        pltpu.VMEM((CH,), _f32),
        pltpu.VMEM((640,), _f32),
        pltpu.VMEM_SHARED((NPAD, DH), _f32),
    ]
    if with_deg:
        scratch.append(pltpu.VMEM_SHARED((NPAD,), _f32))
    scratch += [pltpu.SemaphoreType.DMA for _ in range(2 * NBUF)]
    if with_deg:
        scratch.append(pltpu.SemaphoreType.DMA)
    return pl.kernel(functools.partial(_sc_layer_body, with_deg),
                     out_type=tuple(out_type), mesh=mesh,
                     scratch_types=tuple(scratch),
                     compiler_params=pltpu.CompilerParams(use_tc_tiling_on_sc=False),
                     interpret=interpret)


_BR = 2000


def _dotT(a, w):
    return lax.dot_general(a, w, (((1,), (1,)), ((), ())),
                           preferred_element_type=_f32)


def _tc_pre_body(x_ref, wn_ref, ws_ref, b_ref, hn_ref, hs_ref):
    xb = x_ref[...]
    hn_ref[...] = _dotT(xb, wn_ref[...])
    hs_ref[...] = _dotT(xb, ws_ref[...]) + b_ref[...]


def _tc_pre(x, wn, ws, b, interpret=False):
    grid = (N // _BR,)
    return pl.pallas_call(
        _tc_pre_body,
        grid=grid,
        in_specs=[
            pl.BlockSpec((_BR, DIN), lambda i: (i, 0)),
            pl.BlockSpec((DH, DIN), lambda i: (0, 0)),
            pl.BlockSpec((DH, DIN), lambda i: (0, 0)),
            pl.BlockSpec((1, DH), lambda i: (0, 0)),
        ],
        out_specs=[
            pl.BlockSpec((_BR, DH), lambda i: (i, 0)),
            pl.BlockSpec((_BR, DH), lambda i: (i, 0)),
        ],
        out_shape=[
            jax.ShapeDtypeStruct((N, DH), _f32),
            jax.ShapeDtypeStruct((N, DH), _f32),
        ],
        interpret=interpret,
    )(x, wn, ws, b)


def _tc_mid_body(hs_ref, a0_ref, a1_ref, d0_ref, d1_ref, wn_ref, ws_ref, b_ref,
                 h_ref, hn_ref, hs_out_ref):
    deg = jnp.maximum(d0_ref[0] + d1_ref[0], 1.0)
    h = jnp.tanh(hs_ref[...] + (a0_ref[0] + a1_ref[0]) / deg)
    h_ref[...] = h
    hn_ref[...] = _dotT(h, wn_ref[...])
    hs_out_ref[...] = _dotT(h, ws_ref[...]) + b_ref[...]


def _tc_mid(hs, agg, deg3, wn, ws, b, interpret=False):
    grid = (N // _BR,)
    return pl.pallas_call(
        _tc_mid_body,
        grid=grid,
        in_specs=[
            pl.BlockSpec((_BR, DH), lambda i: (i, 0)),
            pl.BlockSpec((1, _BR, DH), lambda i: (0, i, 0)),
            pl.BlockSpec((1, _BR, DH), lambda i: (1, i, 0)),
            pl.BlockSpec((1, _BR, 1), lambda i: (0, i, 0)),
            pl.BlockSpec((1, _BR, 1), lambda i: (1, i, 0)),
            pl.BlockSpec((DH, DH), lambda i: (0, 0)),
            pl.BlockSpec((DH, DH), lambda i: (0, 0)),
            pl.BlockSpec((1, DH), lambda i: (0, 0)),
        ],
        out_specs=[
            pl.BlockSpec((_BR, DH), lambda i: (i, 0)),
            pl.BlockSpec((_BR, DH), lambda i: (i, 0)),
            pl.BlockSpec((_BR, DH), lambda i: (i, 0)),
        ],
        out_shape=[
            jax.ShapeDtypeStruct((N, DH), _f32),
            jax.ShapeDtypeStruct((N, DH), _f32),
            jax.ShapeDtypeStruct((N, DH), _f32),
        ],
        interpret=interpret,
    )(hs, agg, agg, deg3, deg3, wn, ws, b)


def _tc_head_body(hs3p_ref, a0p_ref, a1p_ref, d0p_ref, d1p_ref,
                  h1p_ref, h2p_ref, h3p_ref, qp_ref,
                  wa_ref, wb_ref, b1_ref, w2_ref, b2_ref, out_ref):
    d0p = d0p_ref[...]
    d1p = d1p_ref[...]
    dege = jnp.maximum(d0p[:, 0:1] + d1p[:, 0:1], 1.0)
    dego = jnp.maximum(d0p[:, 1:2] + d1p[:, 1:2], 1.0)
    hs3p = hs3p_ref[...]
    a0p = a0p_ref[...]
    a1p = a1p_ref[...]
    h4e = jnp.tanh(hs3p[:, 0:DH] + (a0p[:, 0:DH] + a1p[:, 0:DH]) / dege)
    h4o = jnp.tanh(hs3p[:, DH:] + (a0p[:, DH:] + a1p[:, DH:]) / dego)
    qp = qp_ref[...]
    qe = qp[:, 0:1] == 1.0
    qo = qp[:, 1:2] == 1.0
    h1p, h2p, h3p = h1p_ref[...], h2p_ref[...], h3p_ref[...]
    he = [h1p[:, 0:DH], h2p[:, 0:DH], h3p[:, 0:DH], h4e]
    ho = [h1p[:, DH:], h2p[:, DH:], h3p[:, DH:], h4o]
    y = b1_ref[...]
    for i in range(4):
        me = jnp.where(qe, he[i], 0.0)
        mo = jnp.where(qo, ho[i], 0.0)
        y = y + _dotT(me, wa_ref[i])
        y = y + _dotT(mo, wb_ref[i])
    y = jnp.maximum(y, 0.0)
    z = jnp.sum(y * w2_ref[...], axis=1, keepdims=True) + b2_ref[0, 0]
    out_ref[...] = jax.nn.sigmoid(z)


def _tc_head(args, interpret=False):
    full64 = pl.BlockSpec((NPAIR, 2 * DH), lambda: (0, 0))
    full2 = pl.BlockSpec((NPAIR, 2), lambda: (0, 0))
    return pl.pallas_call(
        _tc_head_body,
        in_specs=[full64, full64, full64, full2, full2,
                  full64, full64, full64, full2,
                  pl.BlockSpec((4, DIN, DH), lambda: (0, 0, 0)),
                  pl.BlockSpec((4, DIN, DH), lambda: (0, 0, 0)),
                  pl.BlockSpec((1, DIN), lambda: (0, 0)),
                  pl.BlockSpec((1, DIN), lambda: (0, 0)),
                  pl.BlockSpec(memory_space=pltpu.SMEM)],
        out_specs=pl.BlockSpec((NPAIR, 1), lambda: (0, 0)),
        out_shape=jax.ShapeDtypeStruct((NPAIR, 1), _f32),
        interpret=interpret,
    )(*args)


def kernel(x, edge_index, edge_weight, edge_mask,
           Ws0, bs0, Wn0, bn0, Ws1, bs1, Wn1, bn1,
           Ws2, bs2, Wn2, bn2, Ws3, bs3, Wn3, bn3,
           lin1_W, lin1_b, lin2_W, lin2_b):
    src = edge_index[0]
    dst = edge_index[1]
    ew = edge_weight * edge_mask

    npad_e = EPAD - E
    pad_i = jnp.arange(npad_e, dtype=jnp.int32)
    src_p = jnp.concatenate([src, pad_i % 16]).reshape(NROWS, CH)
    dst_p = jnp.concatenate([dst, N + pad_i % (NPAD - N)]).reshape(NROWS, CH)
    ew_p = jnp.concatenate([ew, jnp.zeros((npad_e,), _f32)]).reshape(NROWS, CH)

    sc_deg = _make_sc_layer(True)
    sc_nodeg = _make_sc_layer(False)

    params = [(Ws0, bs0, Wn0, bn0), (Ws1, bs1, Wn1, bn1),
              (Ws2, bs2, Wn2, bn2), (Ws3, bs3, Wn3, bn3)]

    hn, hs = _tc_pre(x, Wn0, Ws0, (bs0 + bn0).reshape(1, DH))

    states = []
    deg3 = None
    for l in range(4):
        if l == 0:
            agg, deg = sc_deg(hn, src_p, dst_p, ew_p)
            deg3 = deg.reshape(NC, NPAD, 1)
        else:
            (agg,) = sc_nodeg(hn, src_p, dst_p, ew_p)
        if l < 3:
            Ws_n, bs_n, Wn_n, bn_n = params[l + 1]
            h, hn, hs = _tc_mid(hs, agg, deg3, Wn_n, Ws_n,
                                (bs_n + bn_n).reshape(1, DH))
            states.append(h)
        else:
            hs3p = hs[:NQ].reshape(NPAIR, 2 * DH)
            a0p = agg[0, :NQ].reshape(NPAIR, 2 * DH)
            a1p = agg[1, :NQ].reshape(NPAIR, 2 * DH)
            d0p = deg[:NQ].reshape(NPAIR, 2)
            d1p = deg[NPAD:NPAD + NQ].reshape(NPAIR, 2)
            h1p, h2p, h3p = [st[:NQ].reshape(NPAIR, 2 * DH) for st in states]
            qp = x[:NQ, 0:1].reshape(NPAIR, 2)
            wa = jnp.stack([lin1_W[:, 32 * i:32 * i + 32] for i in range(4)])
            wb = jnp.stack([lin1_W[:, 128 + 32 * i:128 + 32 * i + 32]
                            for i in range(4)])
            out = _tc_head((hs3p, a0p, a1p, d0p, d1p, h1p, h2p, h3p, qp,
                            wa, wb, lin1_b.reshape(1, DIN), lin2_W,
                            lin2_b.reshape(1, 1)))
    return out


# speedup vs baseline: 21.1644x; 1.1415x over previous
import functools

import jax
import jax.numpy as jnp
from jax import lax
from jax.experimental import pallas as pl
from jax.experimental.pallas import tpu as pltpu
from jax.experimental.pallas import tpu_sc as plsc

N = 10000
E = 320000
DIN = 128
DH = 32
NQ = 2048
NPAIR = NQ // 2

NC = 2
NS = 16
NW = NC * NS
CH = 128
ROWS_PER_TEC = 80
EPAD = NW * ROWS_PER_TEC * CH
NROWS = EPAD // CH
NPAD = 10112
RPT = NPAD // NS

_f32 = jnp.float32


NBUF = 4


def _sc_layer_body(with_deg, hn_hbm, src_hbm, dst_hbm, ew_hbm, *refs):
    if with_deg:
        agg_out, deg_out = refs[0], refs[1]
        rest = refs[2:]
    else:
        agg_out = refs[0]
        rest = refs[1:]
    src_v, dst_v, ew_v = rest[0], rest[1], rest[2]
    gbuf = list(rest[3:3 + NBUF])
    sbuf = list(rest[3 + NBUF:3 + 2 * NBUF])
    k = 3 + 2 * NBUF
    ones_v, zvec_v, agg_sh, hn_sh = rest[k], rest[k + 1], rest[k + 2], rest[k + 3]
    k += 4
    if with_deg:
        deg_sh = rest[k]
        k += 1
    gsem = list(rest[k:k + NBUF])
    ssem = list(rest[k + NBUF:k + 2 * NBUF])
    if with_deg:
        sem_d = rest[k + 2 * NBUF]
    s0 = sbuf[0]

    c = lax.axis_index("c")
    s = lax.axis_index("s")
    wid = s * NC + c

    zero16 = jnp.zeros((16,), _f32)

    def _zero_rows(i, _):
        s0[i, 0:16] = zero16
        s0[i, 16:32] = zero16
        return 0

    lax.fori_loop(0, CH, _zero_rows, 0)

    def _fill_ones(i, _):
        ones_v[pl.ds(i * 16, 16)] = jnp.ones((16,), _f32)
        return 0

    lax.fori_loop(0, CH // 16, _fill_ones, 0)

    if with_deg:
        def _zero_zvec(i, _):
            zvec_v[pl.ds(i * 16, 16)] = zero16
            return 0

        lax.fori_loop(0, 640 // 16, _zero_zvec, 0)

    base_n = s * RPT
    for k in range(4):
        pltpu.sync_copy(s0, agg_sh.at[pl.ds(base_n + k * CH, CH)])
    pltpu.sync_copy(s0.at[pl.ds(0, RPT - 4 * CH)],
                    agg_sh.at[pl.ds(base_n + 4 * CH, RPT - 4 * CH)])
    if with_deg:
        pltpu.sync_copy(zvec_v.at[pl.ds(0, RPT)], deg_sh.at[pl.ds(base_n, RPT)])

    @pl.when(s < NS - 1)
    def _():
        pltpu.sync_copy(hn_hbm.at[pl.ds(base_n, RPT)],
                        hn_sh.at[pl.ds(base_n, RPT)])

    @pl.when(s == NS - 1)
    def _():
        pltpu.sync_copy(hn_hbm.at[pl.ds((NS - 1) * RPT, N - (NS - 1) * RPT)],
                        hn_sh.at[pl.ds((NS - 1) * RPT, N - (NS - 1) * RPT)])

    plsc.subcore_barrier()

    row0 = wid * ROWS_PER_TEC
    pltpu.sync_copy(src_hbm.at[pl.ds(row0, ROWS_PER_TEC)], src_v)
    pltpu.sync_copy(dst_hbm.at[pl.ds(row0, ROWS_PER_TEC)], dst_v)
    pltpu.sync_copy(ew_hbm.at[pl.ds(row0, ROWS_PER_TEC)], ew_v)

    dn = lax.GatherDimensionNumbers(offset_dims=(), collapsed_slice_dims=(0,),
                                    start_index_map=(0,))

    def _gather(win, b):
        return pltpu.make_async_copy(hn_sh.at[src_v.at[win]], gbuf[b], gsem[b])

    def _scatter_start(b, win):
        pltpu.async_copy(sbuf[b], agg_sh.at[dst_v.at[win]], ssem[b], add=True)

    def _scatter_wait(b):
        pltpu.make_async_copy(sbuf[b], agg_sh.at[dst_v.at[0]], ssem[b]).wait()

    for b in range(NBUF):
        _gather(b, b).start()

    def _outer(i, _):
        j0 = i * NBUF
        for b in range(NBUF):
            j = j0 + b
            _gather(j, b).wait()

            @pl.when(j >= NBUF)
            def _():
                _scatter_wait(b)

            def _scale_group(g_, _c):
                ewv = ew_v[j, pl.ds(g_ * 16, 16)]
                for e in range(16):
                    ewb = lax.gather(ewv, jnp.full((16, 1), e, jnp.int32), dn,
                                     (1,),
                                     mode=lax.GatherScatterMode.PROMISE_IN_BOUNDS)
                    r = g_ * 16 + e
                    sbuf[b][r, 0:16] = gbuf[b][r, 0:16] * ewb
                    sbuf[b][r, 16:32] = gbuf[b][r, 16:32] * ewb
                return 0

            lax.fori_loop(0, CH // 16, _scale_group, 0)

            _scatter_start(b, j)
            if with_deg:
                pltpu.async_copy(ones_v, deg_sh.at[dst_v.at[j]], sem_d, add=True)

                @pl.when(j >= NBUF)
                def _():
                    pltpu.make_async_copy(ones_v, deg_sh.at[dst_v.at[0]],
                                          sem_d).wait()

            @pl.when(j + NBUF < ROWS_PER_TEC)
            def _():
                _gather(j + NBUF, b).start()
        return 0

    lax.fori_loop(0, ROWS_PER_TEC // NBUF, _outer, 0)

    for b in range(NBUF):
        _scatter_wait(b)

    if with_deg:
        for _b in range(NBUF):
            pltpu.make_async_copy(ones_v, deg_sh.at[dst_v.at[0]], sem_d).wait()

    plsc.subcore_barrier()

    pltpu.sync_copy(agg_sh.at[pl.ds(base_n, RPT)], agg_out.at[c, pl.ds(base_n, RPT)])
    if with_deg:
        pltpu.sync_copy(deg_sh.at[pl.ds(base_n, RPT)],
                        deg_out.at[pl.ds(c * NPAD + base_n, RPT)])


def _make_sc_layer(with_deg, interpret=False):
    mesh = plsc.VectorSubcoreMesh(core_axis_name="c", subcore_axis_name="s",
                                  num_cores=NC, num_subcores=NS)
    out_type = [jax.ShapeDtypeStruct((NC, NPAD, DH), _f32)]
    if with_deg:
        out_type.append(jax.ShapeDtypeStruct((NC * NPAD,), _f32))
    scratch = [
        pltpu.VMEM((ROWS_PER_TEC, CH), jnp.int32),
        pltpu.VMEM((ROWS_PER_TEC, CH), jnp.int32),
        pltpu.VMEM((ROWS_PER_TEC, CH), _f32),
    ]
    assert ROWS_PER_TEC % NBUF == 0
    scratch += [pltpu.VMEM((CH, DH), _f32) for _ in range(2 * NBUF)]
    scratch += [
        pltpu.VMEM((CH,), _f32),
        pltpu.VMEM((640,), _f32),
        pltpu.VMEM_SHARED((NPAD, DH), _f32),
        pltpu.VMEM_SHARED((N, DH), _f32),
    ]
    if with_deg:
        scratch.append(pltpu.VMEM_SHARED((NPAD,), _f32))
    scratch += [pltpu.SemaphoreType.DMA for _ in range(2 * NBUF)]
    if with_deg:
        scratch.append(pltpu.SemaphoreType.DMA)
    return pl.kernel(functools.partial(_sc_layer_body, with_deg),
                     out_type=tuple(out_type), mesh=mesh,
                     scratch_types=tuple(scratch),
                     compiler_params=pltpu.CompilerParams(use_tc_tiling_on_sc=False),
                     interpret=interpret)


_BR = 2000


def _dotT(a, w):
    return lax.dot_general(a, w, (((1,), (1,)), ((), ())),
                           preferred_element_type=_f32)


def _tc_pre_body(x_ref, wn_ref, ws_ref, b_ref, hn_ref, hs_ref):
    xb = x_ref[...]
    hn_ref[...] = _dotT(xb, wn_ref[...])
    hs_ref[...] = _dotT(xb, ws_ref[...]) + b_ref[...]


def _tc_pre(x, wn, ws, b, interpret=False):
    grid = (N // _BR,)
    return pl.pallas_call(
        _tc_pre_body,
        grid=grid,
        in_specs=[
            pl.BlockSpec((_BR, DIN), lambda i: (i, 0)),
            pl.BlockSpec((DH, DIN), lambda i: (0, 0)),
            pl.BlockSpec((DH, DIN), lambda i: (0, 0)),
            pl.BlockSpec((1, DH), lambda i: (0, 0)),
        ],
        out_specs=[
            pl.BlockSpec((_BR, DH), lambda i: (i, 0)),
            pl.BlockSpec((_BR, DH), lambda i: (i, 0)),
        ],
        out_shape=[
            jax.ShapeDtypeStruct((N, DH), _f32),
            jax.ShapeDtypeStruct((N, DH), _f32),
        ],
        interpret=interpret,
    )(x, wn, ws, b)


def _tc_mid_body(hs_ref, a0_ref, a1_ref, d0_ref, d1_ref, wn_ref, ws_ref, b_ref,
                 h_ref, hn_ref, hs_out_ref):
    deg = jnp.maximum(d0_ref[0] + d1_ref[0], 1.0)
    h = jnp.tanh(hs_ref[...] + (a0_ref[0] + a1_ref[0]) / deg)
    h_ref[...] = h
    hn_ref[...] = _dotT(h, wn_ref[...])
    hs_out_ref[...] = _dotT(h, ws_ref[...]) + b_ref[...]


def _tc_mid(hs, agg, deg3, wn, ws, b, interpret=False):
    grid = (N // _BR,)
    return pl.pallas_call(
        _tc_mid_body,
        grid=grid,
        in_specs=[
            pl.BlockSpec((_BR, DH), lambda i: (i, 0)),
            pl.BlockSpec((1, _BR, DH), lambda i: (0, i, 0)),
            pl.BlockSpec((1, _BR, DH), lambda i: (1, i, 0)),
            pl.BlockSpec((1, _BR, 1), lambda i: (0, i, 0)),
            pl.BlockSpec((1, _BR, 1), lambda i: (1, i, 0)),
            pl.BlockSpec((DH, DH), lambda i: (0, 0)),
            pl.BlockSpec((DH, DH), lambda i: (0, 0)),
            pl.BlockSpec((1, DH), lambda i: (0, 0)),
        ],
        out_specs=[
            pl.BlockSpec((_BR, DH), lambda i: (i, 0)),
            pl.BlockSpec((_BR, DH), lambda i: (i, 0)),
            pl.BlockSpec((_BR, DH), lambda i: (i, 0)),
        ],
        out_shape=[
            jax.ShapeDtypeStruct((N, DH), _f32),
            jax.ShapeDtypeStruct((N, DH), _f32),
            jax.ShapeDtypeStruct((N, DH), _f32),
        ],
        interpret=interpret,
    )(hs, agg, agg, deg3, deg3, wn, ws, b)


def _tc_head_body(hs3p_ref, a0p_ref, a1p_ref, d0p_ref, d1p_ref,
                  h1p_ref, h2p_ref, h3p_ref, qp_ref,
                  wa_ref, wb_ref, b1_ref, w2_ref, b2_ref, out_ref):
    d0p = d0p_ref[...]
    d1p = d1p_ref[...]
    dege = jnp.maximum(d0p[:, 0:1] + d1p[:, 0:1], 1.0)
    dego = jnp.maximum(d0p[:, 1:2] + d1p[:, 1:2], 1.0)
    hs3p = hs3p_ref[...]
    a0p = a0p_ref[...]
    a1p = a1p_ref[...]
    h4e = jnp.tanh(hs3p[:, 0:DH] + (a0p[:, 0:DH] + a1p[:, 0:DH]) / dege)
    h4o = jnp.tanh(hs3p[:, DH:] + (a0p[:, DH:] + a1p[:, DH:]) / dego)
    qp = qp_ref[...]
    qe = qp[:, 0:1] == 1.0
    qo = qp[:, 1:2] == 1.0
    h1p, h2p, h3p = h1p_ref[...], h2p_ref[...], h3p_ref[...]
    he = [h1p[:, 0:DH], h2p[:, 0:DH], h3p[:, 0:DH], h4e]
    ho = [h1p[:, DH:], h2p[:, DH:], h3p[:, DH:], h4o]
    y = b1_ref[...]
    for i in range(4):
        me = jnp.where(qe, he[i], 0.0)
        mo = jnp.where(qo, ho[i], 0.0)
        y = y + _dotT(me, wa_ref[i])
        y = y + _dotT(mo, wb_ref[i])
    y = jnp.maximum(y, 0.0)
    z = jnp.sum(y * w2_ref[...], axis=1, keepdims=True) + b2_ref[0, 0]
    out_ref[...] = jax.nn.sigmoid(z)


def _tc_head(args, interpret=False):
    full64 = pl.BlockSpec((NPAIR, 2 * DH), lambda: (0, 0))
    full2 = pl.BlockSpec((NPAIR, 2), lambda: (0, 0))
    return pl.pallas_call(
        _tc_head_body,
        in_specs=[full64, full64, full64, full2, full2,
                  full64, full64, full64, full2,
                  pl.BlockSpec((4, DIN, DH), lambda: (0, 0, 0)),
                  pl.BlockSpec((4, DIN, DH), lambda: (0, 0, 0)),
                  pl.BlockSpec((1, DIN), lambda: (0, 0)),
                  pl.BlockSpec((1, DIN), lambda: (0, 0)),
                  pl.BlockSpec(memory_space=pltpu.SMEM)],
        out_specs=pl.BlockSpec((NPAIR, 1), lambda: (0, 0)),
        out_shape=jax.ShapeDtypeStruct((NPAIR, 1), _f32),
        interpret=interpret,
    )(*args)


def kernel(x, edge_index, edge_weight, edge_mask,
           Ws0, bs0, Wn0, bn0, Ws1, bs1, Wn1, bn1,
           Ws2, bs2, Wn2, bn2, Ws3, bs3, Wn3, bn3,
           lin1_W, lin1_b, lin2_W, lin2_b):
    src = edge_index[0]
    dst = edge_index[1]
    ew = edge_weight * edge_mask

    npad_e = EPAD - E
    pad_i = jnp.arange(npad_e, dtype=jnp.int32)
    src_p = jnp.concatenate([src, pad_i % 16]).reshape(NROWS, CH)
    dst_p = jnp.concatenate([dst, N + pad_i % (NPAD - N)]).reshape(NROWS, CH)
    ew_p = jnp.concatenate([ew, jnp.zeros((npad_e,), _f32)]).reshape(NROWS, CH)

    sc_deg = _make_sc_layer(True)
    sc_nodeg = _make_sc_layer(False)

    params = [(Ws0, bs0, Wn0, bn0), (Ws1, bs1, Wn1, bn1),
              (Ws2, bs2, Wn2, bn2), (Ws3, bs3, Wn3, bn3)]

    hn, hs = _tc_pre(x, Wn0, Ws0, (bs0 + bn0).reshape(1, DH))

    states = []
    deg3 = None
    for l in range(4):
        if l == 0:
            agg, deg = sc_deg(hn, src_p, dst_p, ew_p)
            deg3 = deg.reshape(NC, NPAD, 1)
        else:
            (agg,) = sc_nodeg(hn, src_p, dst_p, ew_p)
        if l < 3:
            Ws_n, bs_n, Wn_n, bn_n = params[l + 1]
            h, hn, hs = _tc_mid(hs, agg, deg3, Wn_n, Ws_n,
                                (bs_n + bn_n).reshape(1, DH))
            states.append(h)
        else:
            hs3p = hs[:NQ].reshape(NPAIR, 2 * DH)
            a0p = agg[0, :NQ].reshape(NPAIR, 2 * DH)
            a1p = agg[1, :NQ].reshape(NPAIR, 2 * DH)
            d0p = deg[:NQ].reshape(NPAIR, 2)
            d1p = deg[NPAD:NPAD + NQ].reshape(NPAIR, 2)
            h1p, h2p, h3p = [st[:NQ].reshape(NPAIR, 2 * DH) for st in states]
            qp = x[:NQ, 0:1].reshape(NPAIR, 2)
            wa = jnp.stack([lin1_W[:, 32 * i:32 * i + 32] for i in range(4)])
            wb = jnp.stack([lin1_W[:, 128 + 32 * i:128 + 32 * i + 32]
                            for i in range(4)])
            out = _tc_head((hs3p, a0p, a1p, d0p, d1p, h1p, h2p, h3p, qp,
                            wa, wb, lin1_b.reshape(1, DIN), lin2_W,
                            lin2_b.reshape(1, 1)))
    return out


# speedup vs baseline: 21.6616x; 1.0235x over previous
import functools

import jax
import jax.numpy as jnp
from jax import lax
from jax.experimental import pallas as pl
from jax.experimental.pallas import tpu as pltpu
from jax.experimental.pallas import tpu_sc as plsc

N = 10000
E = 320000
DIN = 128
DH = 32
NQ = 2048
NPAIR = NQ // 2

NC = 2
NS = 16
NW = NC * NS
CH = 128
ROWS_PER_TEC = 80
EPAD = NW * ROWS_PER_TEC * CH
NROWS = EPAD // CH
NPAD = 10112
RPT = NPAD // NS
NPK = N // 4
NPADPK = NPAD // 4

_f32 = jnp.float32


NBUF = 4


def _sc_layer_body(with_deg, hn_hbm, src_hbm, dst_hbm, ew_hbm, *refs):
    if with_deg:
        agg_out, degx_out = refs[0], refs[1]
        rest = refs[2:]
    else:
        agg_out = refs[0]
        rest = refs[1:]
    src_v, dst_v, ew_v = rest[0], rest[1], rest[2]
    gbuf = list(rest[3:3 + NBUF])
    sbuf = list(rest[3 + NBUF:3 + 2 * NBUF])
    k = 3 + 2 * NBUF
    ones_v, zvec_v, agg_sh, hn_sh = rest[k], rest[k + 1], rest[k + 2], rest[k + 3]
    k += 4
    if with_deg:
        deg_sh = rest[k]
        k += 1
    gsem = list(rest[k:k + NBUF])
    ssem = list(rest[k + NBUF:k + 2 * NBUF])
    if with_deg:
        sem_d = rest[k + 2 * NBUF]
    s0 = sbuf[0]

    c = lax.axis_index("c")
    s = lax.axis_index("s")
    wid = s * NC + c

    zero16 = jnp.zeros((16,), _f32)

    def _zero_rows(i, _):
        s0[i, 0:16] = zero16
        s0[i, 16:32] = zero16
        return 0

    lax.fori_loop(0, CH, _zero_rows, 0)

    def _fill_ones(i, _):
        ones_v[pl.ds(i * 16, 16)] = jnp.ones((16,), _f32)
        return 0

    lax.fori_loop(0, CH // 16, _fill_ones, 0)

    if with_deg:
        def _zero_zvec(i, _):
            zvec_v[pl.ds(i * 16, 16)] = zero16
            return 0

        lax.fori_loop(0, 640 // 16, _zero_zvec, 0)

    base_n = s * RPT
    for k in range(4):
        pltpu.sync_copy(s0, agg_sh.at[pl.ds(base_n + k * CH, CH)])
    pltpu.sync_copy(s0.at[pl.ds(0, RPT - 4 * CH)],
                    agg_sh.at[pl.ds(base_n + 4 * CH, RPT - 4 * CH)])
    if with_deg:
        pltpu.sync_copy(zvec_v.at[pl.ds(0, RPT)], deg_sh.at[pl.ds(base_n, RPT)])

    @pl.when(s < NS - 1)
    def _():
        pltpu.sync_copy(hn_hbm.at[pl.ds(base_n, RPT)],
                        hn_sh.at[pl.ds(base_n, RPT)])

    @pl.when(s == NS - 1)
    def _():
        pltpu.sync_copy(hn_hbm.at[pl.ds((NS - 1) * RPT, N - (NS - 1) * RPT)],
                        hn_sh.at[pl.ds((NS - 1) * RPT, N - (NS - 1) * RPT)])

    plsc.subcore_barrier()

    row0 = wid * ROWS_PER_TEC
    pltpu.sync_copy(src_hbm.at[pl.ds(row0, ROWS_PER_TEC)], src_v)
    pltpu.sync_copy(dst_hbm.at[pl.ds(row0, ROWS_PER_TEC)], dst_v)
    pltpu.sync_copy(ew_hbm.at[pl.ds(row0, ROWS_PER_TEC)], ew_v)

    dn = lax.GatherDimensionNumbers(offset_dims=(), collapsed_slice_dims=(0,),
                                    start_index_map=(0,))

    def _gather(win, b):
        return pltpu.make_async_copy(hn_sh.at[src_v.at[win]], gbuf[b], gsem[b])

    def _scatter_start(b, win):
        pltpu.async_copy(sbuf[b], agg_sh.at[dst_v.at[win]], ssem[b], add=True)

    def _scatter_wait(b):
        pltpu.make_async_copy(sbuf[b], agg_sh.at[dst_v.at[0]], ssem[b]).wait()

    for b in range(NBUF):
        _gather(b, b).start()

    def _outer(i, _):
        j0 = i * NBUF
        for b in range(NBUF):
            j = j0 + b
            _gather(j, b).wait()

            @pl.when(j >= NBUF)
            def _():
                _scatter_wait(b)

            def _scale_group(g_, _c):
                ewv = ew_v[j, pl.ds(g_ * 16, 16)]
                for e in range(16):
                    ewb = lax.gather(ewv, jnp.full((16, 1), e, jnp.int32), dn,
                                     (1,),
                                     mode=lax.GatherScatterMode.PROMISE_IN_BOUNDS)
                    r = g_ * 16 + e
                    sbuf[b][r, 0:16] = gbuf[b][r, 0:16] * ewb
                    sbuf[b][r, 16:32] = gbuf[b][r, 16:32] * ewb
                return 0

            lax.fori_loop(0, CH // 16, _scale_group, 0)

            _scatter_start(b, j)
            if with_deg:
                pltpu.async_copy(ones_v, deg_sh.at[dst_v.at[j]], sem_d, add=True)

                @pl.when(j >= NBUF)
                def _():
                    pltpu.make_async_copy(ones_v, deg_sh.at[dst_v.at[0]],
                                          sem_d).wait()

            @pl.when(j + NBUF < ROWS_PER_TEC)
            def _():
                _gather(j + NBUF, b).start()
        return 0

    lax.fori_loop(0, ROWS_PER_TEC // NBUF, _outer, 0)

    for b in range(NBUF):
        _scatter_wait(b)

    if with_deg:
        for _b in range(NBUF):
            pltpu.make_async_copy(ones_v, deg_sh.at[dst_v.at[0]], sem_d).wait()

    plsc.subcore_barrier()

    pltpu.sync_copy(agg_sh.at[pl.ds(base_n, RPT)],
                    agg_out.at[c, pl.ds(base_n, RPT)])
    if with_deg:
        pltpu.sync_copy(deg_sh.at[pl.ds(base_n, RPT)], zvec_v.at[pl.ds(0, RPT)])
        buf = gbuf[0]

        def _expand_chunk(n0, cnt):
            def _grp(g, _):
                dv = zvec_v[pl.ds(n0 + g * 16, 16)]
                ov = ones_v[pl.ds(0, 16)]
                for e in range(16):
                    val = lax.gather(dv, jnp.full((16, 1), e, jnp.int32), dn,
                                     (1,),
                                     mode=lax.GatherScatterMode.PROMISE_IN_BOUNDS)
                    val = val * ov
                    r = g * 16 + e
                    buf[r, 0:16] = val
                    buf[r, 16:32] = val
                return 0

            lax.fori_loop(0, cnt // 16, _grp, 0)
            rem = cnt % 16
            if rem:
                def _tail(g, _):
                    dv = zvec_v[pl.ds(n0 + cnt - 16, 16)]
                    ov = ones_v[pl.ds(0, 16)]
                    for e in range(rem):
                        val = lax.gather(dv, jnp.full((16, 1), 16 - rem + e,
                                                      jnp.int32), dn, (1,),
                                         mode=lax.GatherScatterMode.PROMISE_IN_BOUNDS)
                        val = val * ov
                        r = g + cnt - rem + e
                        buf[r, 0:16] = val
                        buf[r, 16:32] = val
                    return 0

                lax.fori_loop(0, 1, _tail, 0)
            pltpu.sync_copy(buf.at[pl.ds(0, cnt)],
                            degx_out.at[c, pl.ds(base_n + n0, cnt)])

        for cc in range(4):
            _expand_chunk(cc * CH, CH)
        _expand_chunk(4 * CH, RPT - 4 * CH)


def _make_sc_layer(with_deg, interpret=False):
    mesh = plsc.VectorSubcoreMesh(core_axis_name="c", subcore_axis_name="s",
                                  num_cores=NC, num_subcores=NS)
    out_type = [jax.ShapeDtypeStruct((NC, NPAD, DH), _f32)]
    if with_deg:
        out_type.append(jax.ShapeDtypeStruct((NC, NPAD, DH), _f32))
    scratch = [
        pltpu.VMEM((ROWS_PER_TEC, CH), jnp.int32),
        pltpu.VMEM((ROWS_PER_TEC, CH), jnp.int32),
        pltpu.VMEM((ROWS_PER_TEC, CH), _f32),
    ]
    assert ROWS_PER_TEC % NBUF == 0
    scratch += [pltpu.VMEM((CH, DH), _f32) for _ in range(2 * NBUF)]
    scratch += [
        pltpu.VMEM((CH,), _f32),
        pltpu.VMEM((640,), _f32),
        pltpu.VMEM_SHARED((NPAD, DH), _f32),
        pltpu.VMEM_SHARED((N, DH), _f32),
    ]
    if with_deg:
        scratch.append(pltpu.VMEM_SHARED((NPAD,), _f32))
    scratch += [pltpu.SemaphoreType.DMA for _ in range(2 * NBUF)]
    if with_deg:
        scratch.append(pltpu.SemaphoreType.DMA)
    return pl.kernel(functools.partial(_sc_layer_body, with_deg),
                     out_type=tuple(out_type), mesh=mesh,
                     scratch_types=tuple(scratch),
                     compiler_params=pltpu.CompilerParams(use_tc_tiling_on_sc=False),
                     interpret=interpret)


def _dotT(a, w):
    return lax.dot_general(a, w, (((1,), (1,)), ((), ())),
                           preferred_element_type=_f32)


_BR = 2000


def _tc_pre_body(x_ref, wn_ref, ws_ref, b_ref, hn_ref, hs_ref):
    xb = x_ref[...]
    hn_ref[...] = _dotT(xb, wn_ref[...])
    hs_ref[...] = _dotT(xb, ws_ref[...]) + b_ref[...]


def _tc_pre(x, wn, ws, b, interpret=False):
    grid = (N // _BR,)
    return pl.pallas_call(
        _tc_pre_body,
        grid=grid,
        in_specs=[
            pl.BlockSpec((_BR, DIN), lambda i: (i, 0)),
            pl.BlockSpec((DH, DIN), lambda i: (0, 0)),
            pl.BlockSpec((DH, DIN), lambda i: (0, 0)),
            pl.BlockSpec((1, DH), lambda i: (0, 0)),
        ],
        out_specs=[
            pl.BlockSpec((_BR, DH), lambda i: (i, 0)),
            pl.BlockSpec((_BR, DH), lambda i: (i, 0)),
        ],
        out_shape=[
            jax.ShapeDtypeStruct((N, DH), _f32),
            jax.ShapeDtypeStruct((N, DH), _f32),
        ],
        interpret=interpret,
    )(x, wn, ws, b)


def _tc_mid_body(hs_ref, a0_ref, a1_ref, d0_ref, d1_ref, wn_ref, ws_ref, b_ref,
                 h_ref, hn_ref, hs_out_ref):
    deg = jnp.maximum(d0_ref[0] + d1_ref[0], 1.0)
    h = jnp.tanh(hs_ref[...] + (a0_ref[0] + a1_ref[0]) / deg)
    h_ref[...] = h
    hn_ref[...] = _dotT(h, wn_ref[...])
    hs_out_ref[...] = _dotT(h, ws_ref[...]) + b_ref[...]


def _tc_mid(hs, agg, degx, wn, ws, b, interpret=False):
    grid = (N // _BR,)
    return pl.pallas_call(
        _tc_mid_body,
        grid=grid,
        in_specs=[
            pl.BlockSpec((_BR, DH), lambda i: (i, 0)),
            pl.BlockSpec((1, _BR, DH), lambda i: (0, i, 0)),
            pl.BlockSpec((1, _BR, DH), lambda i: (1, i, 0)),
            pl.BlockSpec((1, _BR, DH), lambda i: (0, i, 0)),
            pl.BlockSpec((1, _BR, DH), lambda i: (1, i, 0)),
            pl.BlockSpec((DH, DH), lambda i: (0, 0)),
            pl.BlockSpec((DH, DH), lambda i: (0, 0)),
            pl.BlockSpec((1, DH), lambda i: (0, 0)),
        ],
        out_specs=[
            pl.BlockSpec((_BR, DH), lambda i: (i, 0)),
            pl.BlockSpec((_BR, DH), lambda i: (i, 0)),
            pl.BlockSpec((_BR, DH), lambda i: (i, 0)),
        ],
        out_shape=[
            jax.ShapeDtypeStruct((N, DH), _f32),
            jax.ShapeDtypeStruct((N, DH), _f32),
            jax.ShapeDtypeStruct((N, DH), _f32),
        ],
        interpret=interpret,
    )(hs, agg, agg, degx, degx, wn, ws, b)


def _tc_head_body(hs3p_ref, a0p_ref, a1p_ref, d0p_ref, d1p_ref,
                  h1p_ref, h2p_ref, h3p_ref, qp_ref,
                  wa_ref, wb_ref, b1_ref, w2_ref, b2_ref, out_ref):
    d0p = d0p_ref[...]
    d1p = d1p_ref[...]
    dege = jnp.maximum(d0p[:, 0:1] + d1p[:, 0:1], 1.0)
    dego = jnp.maximum(d0p[:, 1:2] + d1p[:, 1:2], 1.0)
    hs3p = hs3p_ref[...]
    a0p = a0p_ref[...]
    a1p = a1p_ref[...]
    h4e = jnp.tanh(hs3p[:, 0:DH] + (a0p[:, 0:DH] + a1p[:, 0:DH]) / dege)
    h4o = jnp.tanh(hs3p[:, DH:] + (a0p[:, DH:] + a1p[:, DH:]) / dego)
    qp = qp_ref[...]
    qe = qp[:, 0:1] == 1.0
    qo = qp[:, 1:2] == 1.0
    h1p, h2p, h3p = h1p_ref[...], h2p_ref[...], h3p_ref[...]
    he = [h1p[:, 0:DH], h2p[:, 0:DH], h3p[:, 0:DH], h4e]
    ho = [h1p[:, DH:], h2p[:, DH:], h3p[:, DH:], h4o]
    y = b1_ref[...]
    for i in range(4):
        me = jnp.where(qe, he[i], 0.0)
        mo = jnp.where(qo, ho[i], 0.0)
        y = y + _dotT(me, wa_ref[i])
        y = y + _dotT(mo, wb_ref[i])
    y = jnp.maximum(y, 0.0)
    z = jnp.sum(y * w2_ref[...], axis=1, keepdims=True) + b2_ref[0, 0]
    out_ref[...] = jax.nn.sigmoid(z)


def _tc_head(args, interpret=False):
    full64 = pl.BlockSpec((NPAIR, 2 * DH), lambda: (0, 0))
    full2 = pl.BlockSpec((NPAIR, 2), lambda: (0, 0))
    return pl.pallas_call(
        _tc_head_body,
        in_specs=[full64, full64, full64, full2, full2,
                  full64, full64, full64, full2,
                  pl.BlockSpec((4, DIN, DH), lambda: (0, 0, 0)),
                  pl.BlockSpec((4, DIN, DH), lambda: (0, 0, 0)),
                  pl.BlockSpec((1, DIN), lambda: (0, 0)),
                  pl.BlockSpec((1, DIN), lambda: (0, 0)),
                  pl.BlockSpec(memory_space=pltpu.SMEM)],
        out_specs=pl.BlockSpec((NPAIR, 1), lambda: (0, 0)),
        out_shape=jax.ShapeDtypeStruct((NPAIR, 1), _f32),
        interpret=interpret,
    )(*args)


def kernel(x, edge_index, edge_weight, edge_mask,
           Ws0, bs0, Wn0, bn0, Ws1, bs1, Wn1, bn1,
           Ws2, bs2, Wn2, bn2, Ws3, bs3, Wn3, bn3,
           lin1_W, lin1_b, lin2_W, lin2_b):
    import numpy as np

    src = edge_index[0]
    dst = edge_index[1]
    ew = edge_weight * edge_mask

    npad_e = EPAD - E
    pad_np = np.arange(npad_e, dtype=np.int32)
    pad_src = jnp.asarray(pad_np % 16)
    pad_dst = jnp.asarray(N + pad_np % (NPAD - N))
    src_p = jnp.concatenate([src, pad_src]).reshape(NROWS, CH)
    dst_p = jnp.concatenate([dst, pad_dst]).reshape(NROWS, CH)
    ew_p = jnp.concatenate([ew, jnp.zeros((npad_e,), _f32)]).reshape(NROWS, CH)

    sc_deg = _make_sc_layer(True)
    sc_nodeg = _make_sc_layer(False)

    params = [(Ws0, bs0, Wn0, bn0), (Ws1, bs1, Wn1, bn1),
              (Ws2, bs2, Wn2, bn2), (Ws3, bs3, Wn3, bn3)]

    hn, hs = _tc_pre(x, Wn0, Ws0, (bs0 + bn0).reshape(1, DH))

    states = []
    degx = None
    for l in range(4):
        if l == 0:
            agg, degx = sc_deg(hn, src_p, dst_p, ew_p)
        else:
            (agg,) = sc_nodeg(hn, src_p, dst_p, ew_p)
        if l < 3:
            Ws_n, bs_n, Wn_n, bn_n = params[l + 1]
            h, hn, hs = _tc_mid(hs, agg, degx, Wn_n, Ws_n,
                                (bs_n + bn_n).reshape(1, DH))
            states.append(h)
        else:
            hs3p = hs[:NQ].reshape(NPAIR, 2 * DH)
            a0p = agg[0, :NQ].reshape(NPAIR, 2 * DH)
            a1p = agg[1, :NQ].reshape(NPAIR, 2 * DH)
            d0p = degx[0, :NQ, 0].reshape(NPAIR, 2)
            d1p = degx[1, :NQ, 0].reshape(NPAIR, 2)
            h1p, h2p, h3p = [st[:NQ].reshape(NPAIR, 2 * DH) for st in states]
            qp = x[:NQ, 0:1].reshape(NPAIR, 2)
            wa = jnp.stack([lin1_W[:, 32 * i:32 * i + 32] for i in range(4)])
            wb = jnp.stack([lin1_W[:, 128 + 32 * i:128 + 32 * i + 32]
                            for i in range(4)])
            out = _tc_head((hs3p, a0p, a1p, d0p, d1p, h1p, h2p, h3p, qp,
                            wa, wb, lin1_b.reshape(1, DIN), lin2_W,
                            lin2_b.reshape(1, 1)))
    return out
